# baseline (speedup 1.0000x reference)
import functools

import jax
import jax.numpy as jnp
from jax import lax
from jax.experimental import pallas as pl
from jax.experimental.pallas import tpu as pltpu
from jax.experimental.pallas import tpu_sc as plsc

_N = 10000
_E = 160000
_D = 256
_Q = 64
_NQ = 4
_G = 64
_L = 4

_NTILES = 16
_CHUNK = 128
_NCHUNKS = 80
_EDGES_PER_TILE = _CHUNK * _NCHUNKS
_EPAD = _NTILES * _EDGES_PER_TILE
_ACC_ROWS = _N + 8
_ROWS_PER_TILE = 624
_TAIL_ROWS = _N - _NTILES * _ROWS_PER_TILE
_TAIL_Z = _ACC_ROWS - _NTILES * _ROWS_PER_TILE
_ZROWS = _ROWS_PER_TILE // 2

_BLK = 1000
_NBLK = _N // _BLK


_NBUF = 5
_NOUTER = _NCHUNKS // _NBUF
_VPC = _CHUNK // 16


_QD = 8


def _sc_aggregate(hq, src_t, dst_t, with_deg=False):
    mesh = plsc.VectorSubcoreMesh(core_axis_name="c", subcore_axis_name="s")

    scratch = (
        [pltpu.VMEM((_CHUNK,), jnp.int32) for _ in range(_NBUF)]
        + [pltpu.VMEM((_CHUNK, _Q), jnp.float32) for _ in range(_NBUF)]
        + [
            pltpu.VMEM((_NCHUNKS, _CHUNK), jnp.int32),
            pltpu.VMEM((_NCHUNKS, _CHUNK), jnp.int32),
            pltpu.VMEM((_ZROWS, _Q), jnp.float32),
            pltpu.VMEM_SHARED((_ACC_ROWS, _Q), jnp.float32),
        ]
        + [pltpu.SemaphoreType.DMA for _ in range(2 * _NBUF)]
    )
    out_type = jax.ShapeDtypeStruct((_NQ * _N, _Q), jnp.float32)
    if with_deg:
        scratch = scratch + [
            pltpu.VMEM((_CHUNK, _QD), jnp.float32),
            pltpu.VMEM((_ZROWS, _QD), jnp.float32),
            pltpu.VMEM_SHARED((_ACC_ROWS, _QD), jnp.float32),
            pltpu.SemaphoreType.DMA,
        ]
        out_type = (out_type, jax.ShapeDtypeStruct((_N, _QD), jnp.float32))

    @functools.partial(
        pl.kernel,
        out_type=out_type,
        mesh=mesh,
        scratch_types=scratch,
        compiler_params=pltpu.CompilerParams(use_tc_tiling_on_sc=False),
    )
    def body(h_hbm, src_hbm, dst_hbm, *rest):
        if with_deg:
            out_hbm, deg_hbm = rest[0], rest[1]
            scr = rest[2:]
            ones_v, dzbuf, dacc, dsem = scr[-4:]
        else:
            out_hbm = rest[0]
            scr = rest[1:]
        srcq = scr[0:_NBUF]
        rows = scr[_NBUF:2 * _NBUF]
        src_all = scr[2 * _NBUF]
        dst_all = scr[2 * _NBUF + 1]
        zbuf = scr[2 * _NBUF + 2]
        acc = scr[2 * _NBUF + 3]
        gsem = scr[2 * _NBUF + 4:2 * _NBUF + 4 + _NBUF]
        ssem = scr[2 * _NBUF + 4 + _NBUF:2 * _NBUF + 4 + 2 * _NBUF]

        c = lax.axis_index("c")
        s = lax.axis_index("s")
        r0 = s * _ROWS_PER_TILE
        tail0 = _NTILES * _ROWS_PER_TILE

        pltpu.async_copy(src_hbm.at[s], src_all, gsem[0])
        pltpu.async_copy(dst_hbm.at[s], dst_all, gsem[0])

        def zstore(t, _):
            zbuf[t // 4, pl.ds((t % 4) * 16, 16)] = jnp.zeros((16,), jnp.float32)
            return 0

        lax.fori_loop(0, _ZROWS * 4, zstore, 0)
        if with_deg:
            def dfill(t, _):
                dzbuf[t, :] = jnp.zeros((_QD,), jnp.float32)
                return 0

            lax.fori_loop(0, _ZROWS, dfill, 0)

            def ofill(t, _):
                ones_v[t, :] = jnp.ones((_QD,), jnp.float32)
                return 0

            lax.fori_loop(0, _CHUNK, ofill, 0)
            pltpu.sync_copy(dzbuf, dacc.at[pl.ds(r0, _ZROWS)])
            pltpu.sync_copy(dzbuf, dacc.at[pl.ds(r0 + _ZROWS, _ZROWS)])

            @pl.when(s == _NTILES - 1)
            def _():
                pltpu.sync_copy(dzbuf.at[pl.ds(0, _TAIL_Z)],
                                dacc.at[pl.ds(tail0, _TAIL_Z)])

        pltpu.make_async_copy(src_hbm.at[s], src_all, gsem[0]).wait()
        pltpu.make_async_copy(dst_hbm.at[s], dst_all, gsem[0]).wait()

        for p in range(2):
            q = 2 * p + c

            pltpu.sync_copy(zbuf, acc.at[pl.ds(r0, _ZROWS)])
            pltpu.sync_copy(zbuf, acc.at[pl.ds(r0 + _ZROWS, _ZROWS)])

            @pl.when(s == _NTILES - 1)
            def _():
                pltpu.sync_copy(zbuf.at[pl.ds(0, _TAIL_Z)],
                                acc.at[pl.ds(tail0, _TAIL_Z)])

            plsc.subcore_barrier()

            def outer(g2, _):
                gd = []
                for b in range(_NBUF):
                    j = g2 * _NBUF + b

                    @pl.when(g2 > 0)
                    def _(b=b, j=j):
                        pltpu.make_async_copy(
                            rows[b], acc.at[dst_all.at[j]], ssem[b]).wait()
                    for k in range(_VPC):
                        srcq[b][pl.ds(k * 16, 16)] = (
                            src_all[j, pl.ds(k * 16, 16)] + q)
                    gd.append(pltpu.async_copy(h_hbm.at[srcq[b]], rows[b],
                                               gsem[b]))
                for b in range(_NBUF):
                    j = g2 * _NBUF + b
                    gd[b].wait()
                    pltpu.async_copy(rows[b], acc.at[dst_all.at[j]], ssem[b],
                                     add=True)
                    if p == 0 and with_deg:
                        @pl.when(c == 0)
                        def _(j=j):
                            pltpu.async_copy(ones_v, dacc.at[dst_all.at[j]],
                                             dsem, add=True)
                return 0

            lax.fori_loop(0, _NOUTER, outer, 0)
            for b in range(_NBUF):
                pltpu.make_async_copy(
                    rows[b], acc.at[dst_all.at[_NCHUNKS - _NBUF + b]],
                    ssem[b]).wait()
            if p == 0 and with_deg:
                def ddrain(j, _):
                    @pl.when(c == 0)
                    def _():
                        pltpu.make_async_copy(ones_v, dacc.at[dst_all.at[j]],
                                              dsem).wait()
                    return 0

                lax.fori_loop(0, _NCHUNKS, ddrain, 0)
            plsc.subcore_barrier()

            pltpu.sync_copy(
                acc.at[pl.ds(r0, _ROWS_PER_TILE)],
                out_hbm.at[pl.ds(q * _N + r0, _ROWS_PER_TILE)])

            @pl.when(s == _NTILES - 1)
            def _():
                pltpu.sync_copy(acc.at[pl.ds(tail0, _TAIL_ROWS)],
                                out_hbm.at[pl.ds(q * _N + tail0, _TAIL_ROWS)])

            if p == 0 and with_deg:
                @pl.when(c == 0)
                def _():
                    pltpu.sync_copy(dacc.at[pl.ds(r0, _ROWS_PER_TILE)],
                                    deg_hbm.at[pl.ds(r0, _ROWS_PER_TILE)])

                @pl.when((c == 0) & (s == _NTILES - 1))
                def _():
                    pltpu.sync_copy(dacc.at[pl.ds(tail0, _TAIL_ROWS)],
                                    deg_hbm.at[pl.ds(tail0, _TAIL_ROWS)])

    return body(hq, src_t, dst_t)


def _fused_body(t_in_ref, a4_ref, sum_in_ref, sq_in_ref, g_ref, be_ref,
                w1_ref, b1_ref, w2_ref, b2_ref, ep_ref, deg_ref, batch_ref,
                t_ref, sum_ref, sq_ref, pool_ref, cnt_ref):
    i = pl.program_id(0)
    inv_n = 1.0 / _N
    mean = sum_in_ref[0:1, :] * inv_n
    var = sq_in_ref[0:1, :] * inv_n - mean * mean
    a = lax.rsqrt(var + 1e-5) * g_ref[...]
    bb = be_ref[...] - mean * a
    ep = ep_ref[0, 0]
    u = t_in_ref[...] * ep + jnp.concatenate(
        [a4_ref[k] for k in range(_NQ)], axis=1)
    w1eff = jnp.reshape(a, (_D, 1)) * w1_ref[...]
    bvec = jnp.dot(bb, w1_ref[...], preferred_element_type=jnp.float32)
    dvec = jnp.reshape(deg_ref[0, 0], (_BLK, 1)) + ep
    z = jnp.maximum(
        jnp.dot(u, w1eff, preferred_element_type=jnp.float32)
        + dvec * bvec + b1_ref[...], 0.0)
    z = jnp.maximum(
        jnp.dot(z, w2_ref[...], preferred_element_type=jnp.float32)
        + b2_ref[...], 0.0)
    t_ref[...] = z
    ps = jnp.broadcast_to(jnp.sum(z, axis=0, keepdims=True), (8, _D))
    pq = jnp.broadcast_to(jnp.sum(z * z, axis=0, keepdims=True), (8, _D))
    b = batch_ref[0, 0]
    oh = (b[:, None] == lax.broadcasted_iota(jnp.int32, (_BLK, _G), 1)
          ).astype(jnp.float32)
    pp = lax.dot_general(oh, z, (((0,), (0,)), ((), ())),
                         preferred_element_type=jnp.float32)
    pc = lax.dot_general(oh, jnp.ones((_BLK, 128), jnp.float32),
                         (((0,), (0,)), ((), ())),
                         preferred_element_type=jnp.float32)

    @pl.when(i == 0)
    def _():
        sum_ref[...] = jnp.zeros_like(sum_ref)
        sq_ref[...] = jnp.zeros_like(sq_ref)
        pool_ref[...] = jnp.zeros_like(pool_ref)
        cnt_ref[...] = jnp.zeros_like(cnt_ref)

    sum_ref[...] += ps
    sq_ref[...] += pq
    pool_ref[...] += pp
    cnt_ref[...] += pc


def _fused(t_in, a4, ssum_in, sq_in, gamma, beta, w1, b1, w2, b2, onep,
           deg_r, batch_r):
    full = lambda i: (0, 0)
    return pl.pallas_call(
        _fused_body,
        grid=(_NBLK,),
        in_specs=[
            pl.BlockSpec((_BLK, _D), lambda i: (i, 0)),
            pl.BlockSpec((_NQ, _BLK, _Q), lambda i: (0, i, 0)),
            pl.BlockSpec((8, _D), full),
            pl.BlockSpec((8, _D), full),
            pl.BlockSpec((1, _D), full),
            pl.BlockSpec((1, _D), full),
            pl.BlockSpec((_D, _D), full),
            pl.BlockSpec((1, _D), full),
            pl.BlockSpec((_D, _D), full),
            pl.BlockSpec((1, _D), full),
            pl.BlockSpec((1, 1), full),
            pl.BlockSpec((1, 1, _BLK), lambda i: (i, 0, 0)),
            pl.BlockSpec((1, 1, _BLK), lambda i: (i, 0, 0)),
        ],
        out_specs=[
            pl.BlockSpec((_BLK, _D), lambda i: (i, 0)),
            pl.BlockSpec((8, _D), full),
            pl.BlockSpec((8, _D), full),
            pl.BlockSpec((_G, _D), full),
            pl.BlockSpec((_G, 128), full),
        ],
        out_shape=[
            jax.ShapeDtypeStruct((_N, _D), jnp.float32),
            jax.ShapeDtypeStruct((8, _D), jnp.float32),
            jax.ShapeDtypeStruct((8, _D), jnp.float32),
            jax.ShapeDtypeStruct((_G, _D), jnp.float32),
            jax.ShapeDtypeStruct((_G, 128), jnp.float32),
        ],
        compiler_params=pltpu.CompilerParams(
            dimension_semantics=("arbitrary",)),
    )(t_in, a4, ssum_in, sq_in, gamma, beta, w1, b1, w2, b2, onep,
      deg_r, batch_r)


def _head_body(p0, p1, p2, p3, s0, q0, g0, e0, s1, q1, g1, e1,
               s2, q2, g2, e2, s3, q3, g3, e3, cnt,
               w1, b1, w2, b2, w3, b3, w4, b4, out):
    inv_n = 1.0 / _N
    cnt1 = cnt[:, 0:1]
    phs = []
    for pt, ss, qq, gg, ee in ((p0, s0, q0, g0, e0), (p1, s1, q1, g1, e1),
                               (p2, s2, q2, g2, e2), (p3, s3, q3, g3, e3)):
        mean = ss[0:1, :] * inv_n
        var = qq[0:1, :] * inv_n - mean * mean
        a = lax.rsqrt(var + 1e-5) * gg[...]
        bb = ee[...] - mean * a
        phs.append(pt[...] * a + cnt1 * bb)
    ge = jnp.concatenate(phs, axis=1)
    ge = ge / jnp.maximum(cnt1, 1.0)
    g = jnp.maximum(
        jnp.dot(ge, w1[...], preferred_element_type=jnp.float32) + b1[...], 0.0)
    g = jnp.maximum(
        jnp.dot(g, w2[...], preferred_element_type=jnp.float32) + b2[...], 0.0)
    g = jnp.maximum(
        jnp.dot(g, w3[...], preferred_element_type=jnp.float32) + b3[...], 0.0)
    lg = jnp.dot(g, w4[...], preferred_element_type=jnp.float32) + b4[...]
    m = jnp.max(lg, axis=1, keepdims=True)
    e = jnp.exp(lg - m)
    out[...] = lg - m - jnp.log(jnp.sum(e, axis=1, keepdims=True))


def _head(pools, stats, cnt, c):
    nc = c['W4'].shape[1]
    w4p = jnp.pad(c['W4'], ((0, 0), (0, 128 - nc)))
    b4p = jnp.pad(c['b4'].reshape(1, -1), ((0, 0), (0, 128 - nc)),
                  constant_values=-1e30)
    args = list(pools)
    flat_stats = []
    for st in stats:
        flat_stats.extend(st)
    out = pl.pallas_call(
        _head_body,
        out_shape=jax.ShapeDtypeStruct((_G, 128), jnp.float32),
    )(*(args + flat_stats + [cnt]),
      c['W1'], c['b1'].reshape(1, -1),
      c['W2'], c['b2'].reshape(1, -1),
      c['W3'], c['b3'].reshape(1, -1),
      w4p, b4p)
    return out[:, :nc]


def kernel(x, edge_index, batch, params):
    src = edge_index[0]
    dst = edge_index[1]
    npad = _EPAD - _E
    src_t = jnp.pad(src * _NQ, (0, npad)).reshape(_NTILES, _NCHUNKS, _CHUNK)
    dst_t = jnp.pad(dst, (0, npad), constant_values=_N).reshape(
        _NTILES, _NCHUNKS, _CHUNK)
    batch_r = batch.reshape(_NBLK, 1, _BLK)

    ssum = jnp.zeros((8, _D), jnp.float32)
    sq = jnp.full((8, _D), _N * (1.0 - 1e-5), jnp.float32)
    gamma = jnp.ones((1, _D), jnp.float32)
    beta = jnp.zeros((1, _D), jnp.float32)

    t = x
    pools = []
    stats = []
    cnt = None
    deg_r = None
    for li in range(_L):
        p = params['gin%d' % li]
        if li == 0:
            aggr, deg = _sc_aggregate(t.reshape(_NQ * _N, _Q), src_t, dst_t,
                                      with_deg=True)
            deg_r = deg[:, 0].reshape(_NBLK, 1, _BLK)
        else:
            aggr = _sc_aggregate(t.reshape(_NQ * _N, _Q), src_t, dst_t)
        a4 = aggr.reshape(_NQ, _N, _Q)
        onep = (1.0 + p['eps']).reshape(1, 1)
        t, ssum, sq, pool_i, cnt_i = _fused(
            t, a4, ssum, sq, gamma, beta,
            p['W1'], p['b1'].reshape(1, -1),
            p['W2'], p['b2'].reshape(1, -1), onep, deg_r, batch_r)
        gamma = p['gamma'].reshape(1, -1)
        beta = p['beta'].reshape(1, -1)
        pools.append(pool_i)
        stats.append((ssum, sq, gamma, beta))
        if cnt is None:
            cnt = cnt_i

    return _head(pools, stats, cnt, params['cls'])


# speedup vs baseline: 2.5291x; 2.5291x over previous
import functools

import jax
import jax.numpy as jnp
from jax import lax
from jax.experimental import pallas as pl
from jax.experimental.pallas import tpu as pltpu
from jax.experimental.pallas import tpu_sc as plsc

_N = 10000
_E = 160000
_D = 256
_Q = 64
_NQ = 4
_G = 64
_L = 4

_NTILES = 16
_CHUNK = 80
_NCHUNKS = 125
_EDGES_PER_TILE = _CHUNK * _NCHUNKS
_EPAD = _NTILES * _EDGES_PER_TILE
_ACC_ROWS = _N + 8
_ROWS_PER_TILE = 624
_TAIL_ROWS = _N - _NTILES * _ROWS_PER_TILE
_TAIL_Z = _ACC_ROWS - _NTILES * _ROWS_PER_TILE
_ZROWS = _ROWS_PER_TILE // 2

_BLK = 1000
_NBLK = _N // _BLK


_NBUF = 5
_NOUTER = _NCHUNKS // _NBUF
_VPC = _CHUNK // 16


_QD = 8


def _sc_aggregate(hq, src_t, dst_t, with_deg=False):
    mesh = plsc.VectorSubcoreMesh(core_axis_name="c", subcore_axis_name="s")

    scratch = (
        [pltpu.VMEM((_CHUNK,), jnp.int32) for _ in range(_NBUF)]
        + [pltpu.VMEM((_CHUNK, _Q), jnp.float32) for _ in range(_NBUF)]
        + [
            pltpu.VMEM((_NCHUNKS, _CHUNK), jnp.int32),
            pltpu.VMEM((_NCHUNKS, _CHUNK), jnp.int32),
            pltpu.VMEM((_ZROWS, _Q), jnp.float32),
            pltpu.VMEM_SHARED((_ACC_ROWS, _Q), jnp.float32),
        ]
        + [pltpu.SemaphoreType.DMA for _ in range(2 * _NBUF)]
    )
    out_type = jax.ShapeDtypeStruct((_NQ * _N, _Q), jnp.float32)
    if with_deg:
        scratch = scratch + [
            pltpu.VMEM((_CHUNK, _QD), jnp.float32),
            pltpu.VMEM((_ZROWS, _QD), jnp.float32),
            pltpu.VMEM_SHARED((_ACC_ROWS, _QD), jnp.float32),
            pltpu.SemaphoreType.DMA,
        ]
        out_type = (out_type, jax.ShapeDtypeStruct((_N, _QD), jnp.float32))

    @functools.partial(
        pl.kernel,
        out_type=out_type,
        mesh=mesh,
        scratch_types=scratch,
        compiler_params=pltpu.CompilerParams(use_tc_tiling_on_sc=False),
    )
    def body(h_hbm, src_hbm, dst_hbm, *rest):
        if with_deg:
            out_hbm, deg_hbm = rest[0], rest[1]
            scr = rest[2:]
            ones_v, dzbuf, dacc, dsem = scr[-4:]
        else:
            out_hbm = rest[0]
            scr = rest[1:]
        srcq = scr[0:_NBUF]
        rows = scr[_NBUF:2 * _NBUF]
        src_all = scr[2 * _NBUF]
        dst_all = scr[2 * _NBUF + 1]
        zbuf = scr[2 * _NBUF + 2]
        acc = scr[2 * _NBUF + 3]
        gsem = scr[2 * _NBUF + 4:2 * _NBUF + 4 + _NBUF]
        ssem = scr[2 * _NBUF + 4 + _NBUF:2 * _NBUF + 4 + 2 * _NBUF]

        c = lax.axis_index("c")
        s = lax.axis_index("s")
        r0 = s * _ROWS_PER_TILE
        tail0 = _NTILES * _ROWS_PER_TILE

        pltpu.async_copy(src_hbm.at[s], src_all, gsem[0])
        pltpu.async_copy(dst_hbm.at[s], dst_all, gsem[0])

        def zstore(t, _):
            zbuf[t // 4, pl.ds((t % 4) * 16, 16)] = jnp.zeros((16,), jnp.float32)
            return 0

        lax.fori_loop(0, _ZROWS * 4, zstore, 0)
        if with_deg:
            def dfill(t, _):
                dzbuf[t, :] = jnp.zeros((_QD,), jnp.float32)
                return 0

            lax.fori_loop(0, _ZROWS, dfill, 0)

            def ofill(t, _):
                ones_v[t, :] = jnp.ones((_QD,), jnp.float32)
                return 0

            lax.fori_loop(0, _CHUNK, ofill, 0)
            pltpu.sync_copy(dzbuf, dacc.at[pl.ds(r0, _ZROWS)])
            pltpu.sync_copy(dzbuf, dacc.at[pl.ds(r0 + _ZROWS, _ZROWS)])

            @pl.when(s == _NTILES - 1)
            def _():
                pltpu.sync_copy(dzbuf.at[pl.ds(0, _TAIL_Z)],
                                dacc.at[pl.ds(tail0, _TAIL_Z)])

        pltpu.make_async_copy(src_hbm.at[s], src_all, gsem[0]).wait()
        pltpu.make_async_copy(dst_hbm.at[s], dst_all, gsem[0]).wait()

        for p in range(2):
            q = 2 * p + c

            pltpu.sync_copy(zbuf, acc.at[pl.ds(r0, _ZROWS)])
            pltpu.sync_copy(zbuf, acc.at[pl.ds(r0 + _ZROWS, _ZROWS)])

            @pl.when(s == _NTILES - 1)
            def _():
                pltpu.sync_copy(zbuf.at[pl.ds(0, _TAIL_Z)],
                                acc.at[pl.ds(tail0, _TAIL_Z)])

            plsc.subcore_barrier()

            def outer(g2, _):
                gd = []
                for b in range(_NBUF):
                    j = g2 * _NBUF + b

                    @pl.when(g2 > 0)
                    def _(b=b, j=j):
                        pltpu.make_async_copy(
                            rows[b], acc.at[dst_all.at[j]], ssem[b]).wait()
                    for k in range(_VPC):
                        srcq[b][pl.ds(k * 16, 16)] = (
                            src_all[j, pl.ds(k * 16, 16)] + q)
                    gd.append(pltpu.async_copy(h_hbm.at[srcq[b]], rows[b],
                                               gsem[b]))
                for b in range(_NBUF):
                    j = g2 * _NBUF + b
                    gd[b].wait()
                    pltpu.async_copy(rows[b], acc.at[dst_all.at[j]], ssem[b],
                                     add=True)
                    if p == 0 and with_deg:
                        @pl.when(c == 0)
                        def _(j=j):
                            pltpu.async_copy(ones_v, dacc.at[dst_all.at[j]],
                                             dsem, add=True)
                return 0

            lax.fori_loop(0, _NOUTER, outer, 0)
            for b in range(_NBUF):
                pltpu.make_async_copy(
                    rows[b], acc.at[dst_all.at[_NCHUNKS - _NBUF + b]],
                    ssem[b]).wait()
            if p == 0 and with_deg:
                def ddrain(j, _):
                    @pl.when(c == 0)
                    def _():
                        pltpu.make_async_copy(ones_v, dacc.at[dst_all.at[j]],
                                              dsem).wait()
                    return 0

                lax.fori_loop(0, _NCHUNKS, ddrain, 0)
            plsc.subcore_barrier()

            pltpu.sync_copy(
                acc.at[pl.ds(r0, _ROWS_PER_TILE)],
                out_hbm.at[pl.ds(q * _N + r0, _ROWS_PER_TILE)])

            @pl.when(s == _NTILES - 1)
            def _():
                pltpu.sync_copy(acc.at[pl.ds(tail0, _TAIL_ROWS)],
                                out_hbm.at[pl.ds(q * _N + tail0, _TAIL_ROWS)])

            if p == 0 and with_deg:
                @pl.when(c == 0)
                def _():
                    pltpu.sync_copy(dacc.at[pl.ds(r0, _ROWS_PER_TILE)],
                                    deg_hbm.at[pl.ds(r0, _ROWS_PER_TILE)])

                @pl.when((c == 0) & (s == _NTILES - 1))
                def _():
                    pltpu.sync_copy(dacc.at[pl.ds(tail0, _TAIL_ROWS)],
                                    deg_hbm.at[pl.ds(tail0, _TAIL_ROWS)])

    return body(hq, src_t, dst_t)


def _fused_body(t_in_ref, a4_ref, sum_in_ref, sq_in_ref, g_ref, be_ref,
                w1_ref, b1_ref, w2_ref, b2_ref, ep_ref, deg_ref, batch_ref,
                t_ref, sum_ref, sq_ref, pool_ref, cnt_ref):
    i = pl.program_id(0)
    inv_n = 1.0 / _N
    mean = sum_in_ref[0:1, :] * inv_n
    var = sq_in_ref[0:1, :] * inv_n - mean * mean
    a = lax.rsqrt(var + 1e-5) * g_ref[...]
    bb = be_ref[...] - mean * a
    ep = ep_ref[0, 0]
    u = t_in_ref[...] * ep + jnp.concatenate(
        [a4_ref[k] for k in range(_NQ)], axis=1)
    w1eff = jnp.reshape(a, (_D, 1)) * w1_ref[...]
    bvec = jnp.dot(bb, w1_ref[...], preferred_element_type=jnp.float32)
    dvec = jnp.reshape(deg_ref[0, 0], (_BLK, 1)) + ep
    z = jnp.maximum(
        jnp.dot(u, w1eff, preferred_element_type=jnp.float32)
        + dvec * bvec + b1_ref[...], 0.0)
    z = jnp.maximum(
        jnp.dot(z, w2_ref[...], preferred_element_type=jnp.float32)
        + b2_ref[...], 0.0)
    t_ref[...] = z
    ps = jnp.broadcast_to(jnp.sum(z, axis=0, keepdims=True), (8, _D))
    pq = jnp.broadcast_to(jnp.sum(z * z, axis=0, keepdims=True), (8, _D))
    b = batch_ref[0, 0]
    oh = (b[:, None] == lax.broadcasted_iota(jnp.int32, (_BLK, _G), 1)
          ).astype(jnp.float32)
    pp = lax.dot_general(oh, z, (((0,), (0,)), ((), ())),
                         preferred_element_type=jnp.float32)
    pc = lax.dot_general(oh, jnp.ones((_BLK, 128), jnp.float32),
                         (((0,), (0,)), ((), ())),
                         preferred_element_type=jnp.float32)

    @pl.when(i == 0)
    def _():
        sum_ref[...] = jnp.zeros_like(sum_ref)
        sq_ref[...] = jnp.zeros_like(sq_ref)
        pool_ref[...] = jnp.zeros_like(pool_ref)
        cnt_ref[...] = jnp.zeros_like(cnt_ref)

    sum_ref[...] += ps
    sq_ref[...] += pq
    pool_ref[...] += pp
    cnt_ref[...] += pc


def _fused(t_in, a4, ssum_in, sq_in, gamma, beta, w1, b1, w2, b2, onep,
           deg_r, batch_r):
    full = lambda i: (0, 0)
    return pl.pallas_call(
        _fused_body,
        grid=(_NBLK,),
        in_specs=[
            pl.BlockSpec((_BLK, _D), lambda i: (i, 0)),
            pl.BlockSpec((_NQ, _BLK, _Q), lambda i: (0, i, 0)),
            pl.BlockSpec((8, _D), full),
            pl.BlockSpec((8, _D), full),
            pl.BlockSpec((1, _D), full),
            pl.BlockSpec((1, _D), full),
            pl.BlockSpec((_D, _D), full),
            pl.BlockSpec((1, _D), full),
            pl.BlockSpec((_D, _D), full),
            pl.BlockSpec((1, _D), full),
            pl.BlockSpec((1, 1), full),
            pl.BlockSpec((1, 1, _BLK), lambda i: (i, 0, 0)),
            pl.BlockSpec((1, 1, _BLK), lambda i: (i, 0, 0)),
        ],
        out_specs=[
            pl.BlockSpec((_BLK, _D), lambda i: (i, 0)),
            pl.BlockSpec((8, _D), full),
            pl.BlockSpec((8, _D), full),
            pl.BlockSpec((_G, _D), full),
            pl.BlockSpec((_G, 128), full),
        ],
        out_shape=[
            jax.ShapeDtypeStruct((_N, _D), jnp.float32),
            jax.ShapeDtypeStruct((8, _D), jnp.float32),
            jax.ShapeDtypeStruct((8, _D), jnp.float32),
            jax.ShapeDtypeStruct((_G, _D), jnp.float32),
            jax.ShapeDtypeStruct((_G, 128), jnp.float32),
        ],
        compiler_params=pltpu.CompilerParams(
            dimension_semantics=("arbitrary",)),
    )(t_in, a4, ssum_in, sq_in, gamma, beta, w1, b1, w2, b2, onep,
      deg_r, batch_r)


def _head_body(p0, p1, p2, p3, s0, q0, g0, e0, s1, q1, g1, e1,
               s2, q2, g2, e2, s3, q3, g3, e3, cnt,
               w1, b1, w2, b2, w3, b3, w4, b4, out):
    inv_n = 1.0 / _N
    cnt1 = cnt[:, 0:1]
    phs = []
    for pt, ss, qq, gg, ee in ((p0, s0, q0, g0, e0), (p1, s1, q1, g1, e1),
                               (p2, s2, q2, g2, e2), (p3, s3, q3, g3, e3)):
        mean = ss[0:1, :] * inv_n
        var = qq[0:1, :] * inv_n - mean * mean
        a = lax.rsqrt(var + 1e-5) * gg[...]
        bb = ee[...] - mean * a
        phs.append(pt[...] * a + cnt1 * bb)
    ge = jnp.concatenate(phs, axis=1)
    ge = ge / jnp.maximum(cnt1, 1.0)
    g = jnp.maximum(
        jnp.dot(ge, w1[...], preferred_element_type=jnp.float32) + b1[...], 0.0)
    g = jnp.maximum(
        jnp.dot(g, w2[...], preferred_element_type=jnp.float32) + b2[...], 0.0)
    g = jnp.maximum(
        jnp.dot(g, w3[...], preferred_element_type=jnp.float32) + b3[...], 0.0)
    lg = jnp.dot(g, w4[...], preferred_element_type=jnp.float32) + b4[...]
    m = jnp.max(lg, axis=1, keepdims=True)
    e = jnp.exp(lg - m)
    out[...] = lg - m - jnp.log(jnp.sum(e, axis=1, keepdims=True))


def _head(pools, stats, cnt, c):
    nc = c['W4'].shape[1]
    w4p = jnp.pad(c['W4'], ((0, 0), (0, 128 - nc)))
    b4p = jnp.pad(c['b4'].reshape(1, -1), ((0, 0), (0, 128 - nc)),
                  constant_values=-1e30)
    args = list(pools)
    flat_stats = []
    for st in stats:
        flat_stats.extend(st)
    out = pl.pallas_call(
        _head_body,
        out_shape=jax.ShapeDtypeStruct((_G, 128), jnp.float32),
    )(*(args + flat_stats + [cnt]),
      c['W1'], c['b1'].reshape(1, -1),
      c['W2'], c['b2'].reshape(1, -1),
      c['W3'], c['b3'].reshape(1, -1),
      w4p, b4p)
    return out[:, :nc]


def kernel(x, edge_index, batch, params):
    src = edge_index[0]
    dst = edge_index[1]
    npad = _EPAD - _E
    src_t = jnp.pad(src * _NQ, (0, npad)).reshape(_NTILES, _NCHUNKS, _CHUNK)
    dst_t = jnp.pad(dst, (0, npad), constant_values=_N).reshape(
        _NTILES, _NCHUNKS, _CHUNK)
    batch_r = batch.reshape(_NBLK, 1, _BLK)

    ssum = jnp.zeros((8, _D), jnp.float32)
    sq = jnp.full((8, _D), _N * (1.0 - 1e-5), jnp.float32)
    gamma = jnp.ones((1, _D), jnp.float32)
    beta = jnp.zeros((1, _D), jnp.float32)

    t = x
    pools = []
    stats = []
    cnt = None
    deg_r = None
    for li in range(_L):
        p = params['gin%d' % li]
        if li == 0:
            aggr, deg = _sc_aggregate(t.reshape(_NQ * _N, _Q), src_t, dst_t,
                                      with_deg=True)
            deg_r = deg[:, 0].reshape(_NBLK, 1, _BLK)
        else:
            aggr = _sc_aggregate(t.reshape(_NQ * _N, _Q), src_t, dst_t)
        a4 = aggr.reshape(_NQ, _N, _Q)
        onep = (1.0 + p['eps']).reshape(1, 1)
        t, ssum, sq, pool_i, cnt_i = _fused(
            t, a4, ssum, sq, gamma, beta,
            p['W1'], p['b1'].reshape(1, -1),
            p['W2'], p['b2'].reshape(1, -1), onep, deg_r, batch_r)
        gamma = p['gamma'].reshape(1, -1)
        beta = p['beta'].reshape(1, -1)
        pools.append(pool_i)
        stats.append((ssum, sq, gamma, beta))
        if cnt is None:
            cnt = cnt_i

    return _head(pools, stats, cnt, params['cls'])


# speedup vs baseline: 2.5327x; 1.0014x over previous
import functools

import jax
import jax.numpy as jnp
from jax import lax
from jax.experimental import pallas as pl
from jax.experimental.pallas import tpu as pltpu
from jax.experimental.pallas import tpu_sc as plsc

_N = 10000
_E = 160000
_D = 256
_Q = 64
_NQ = 4
_G = 64
_L = 4

_NTILES = 16
_CHUNK = 80
_NCHUNKS = 125
_EDGES_PER_TILE = _CHUNK * _NCHUNKS
_EPAD = _NTILES * _EDGES_PER_TILE
_ACC_ROWS = _N + 8
_ROWS_PER_TILE = 624
_TAIL_ROWS = _N - _NTILES * _ROWS_PER_TILE
_TAIL_Z = _ACC_ROWS - _NTILES * _ROWS_PER_TILE
_ZROWS = _ROWS_PER_TILE // 2

_BLK = 1000
_NBLK = _N // _BLK


_NBUF = 5
_NOUTER = _NCHUNKS // _NBUF
_VPC = _CHUNK // 16


_QD = 8


def _sc_aggregate(hq, src_t, dst_t, with_deg=False):
    mesh = plsc.VectorSubcoreMesh(core_axis_name="c", subcore_axis_name="s")

    scratch = (
        [pltpu.VMEM((_CHUNK,), jnp.int32) for _ in range(_NBUF)]
        + [pltpu.VMEM((_CHUNK, _Q), jnp.float32) for _ in range(_NBUF)]
        + [
            pltpu.VMEM((_NCHUNKS, _CHUNK), jnp.int32),
            pltpu.VMEM((_NCHUNKS, _CHUNK), jnp.int32),
            pltpu.VMEM((_ZROWS, _Q), jnp.float32),
            pltpu.VMEM_SHARED((_ACC_ROWS, _Q), jnp.float32),
        ]
        + [pltpu.SemaphoreType.DMA for _ in range(2 * _NBUF)]
    )
    out_type = jax.ShapeDtypeStruct((_NQ * _N, _Q), jnp.float32)
    if with_deg:
        scratch = scratch + [
            pltpu.VMEM((_CHUNK, _QD), jnp.float32),
            pltpu.VMEM((_ZROWS, _QD), jnp.float32),
            pltpu.VMEM_SHARED((_ACC_ROWS, _QD), jnp.float32),
            pltpu.SemaphoreType.DMA,
        ]
        out_type = (out_type, jax.ShapeDtypeStruct((_N, _QD), jnp.float32))

    @functools.partial(
        pl.kernel,
        out_type=out_type,
        mesh=mesh,
        scratch_types=scratch,
        compiler_params=pltpu.CompilerParams(use_tc_tiling_on_sc=False),
    )
    def body(h_hbm, src_hbm, dst_hbm, *rest):
        if with_deg:
            out_hbm, deg_hbm = rest[0], rest[1]
            scr = rest[2:]
            ones_v, dzbuf, dacc, dsem = scr[-4:]
        else:
            out_hbm = rest[0]
            scr = rest[1:]
        srcq = scr[0:_NBUF]
        rows = scr[_NBUF:2 * _NBUF]
        src_all = scr[2 * _NBUF]
        dst_all = scr[2 * _NBUF + 1]
        zbuf = scr[2 * _NBUF + 2]
        acc = scr[2 * _NBUF + 3]
        gsem = scr[2 * _NBUF + 4:2 * _NBUF + 4 + _NBUF]
        ssem = scr[2 * _NBUF + 4 + _NBUF:2 * _NBUF + 4 + 2 * _NBUF]

        c = lax.axis_index("c")
        s = lax.axis_index("s")
        r0 = s * _ROWS_PER_TILE
        tail0 = _NTILES * _ROWS_PER_TILE

        pltpu.async_copy(src_hbm.at[s], src_all, gsem[0])
        pltpu.async_copy(dst_hbm.at[s], dst_all, gsem[0])

        def zstore(t, _):
            zbuf[t // 4, pl.ds((t % 4) * 16, 16)] = jnp.zeros((16,), jnp.float32)
            return 0

        lax.fori_loop(0, _ZROWS * 4, zstore, 0)
        if with_deg:
            def dfill(t, _):
                dzbuf[t, :] = jnp.zeros((_QD,), jnp.float32)
                return 0

            lax.fori_loop(0, _ZROWS, dfill, 0)

            def ofill(t, _):
                ones_v[t, :] = jnp.ones((_QD,), jnp.float32)
                return 0

            lax.fori_loop(0, _CHUNK, ofill, 0)
            pltpu.sync_copy(dzbuf, dacc.at[pl.ds(r0, _ZROWS)])
            pltpu.sync_copy(dzbuf, dacc.at[pl.ds(r0 + _ZROWS, _ZROWS)])

            @pl.when(s == _NTILES - 1)
            def _():
                pltpu.sync_copy(dzbuf.at[pl.ds(0, _TAIL_Z)],
                                dacc.at[pl.ds(tail0, _TAIL_Z)])

        pltpu.make_async_copy(src_hbm.at[s], src_all, gsem[0]).wait()
        pltpu.make_async_copy(dst_hbm.at[s], dst_all, gsem[0]).wait()

        for p in range(2):
            q = 2 * p + c

            pltpu.sync_copy(zbuf, acc.at[pl.ds(r0, _ZROWS)])
            pltpu.sync_copy(zbuf, acc.at[pl.ds(r0 + _ZROWS, _ZROWS)])

            @pl.when(s == _NTILES - 1)
            def _():
                pltpu.sync_copy(zbuf.at[pl.ds(0, _TAIL_Z)],
                                acc.at[pl.ds(tail0, _TAIL_Z)])

            plsc.subcore_barrier()

            def outer(g2, _):
                gd = []
                for b in range(_NBUF):
                    j = g2 * _NBUF + b

                    @pl.when(g2 > 0)
                    def _(b=b, j=j):
                        pltpu.make_async_copy(
                            rows[b], acc.at[dst_all.at[j]], ssem[b]).wait()
                    for k in range(_VPC):
                        srcq[b][pl.ds(k * 16, 16)] = (
                            src_all[j, pl.ds(k * 16, 16)] + q)
                    gd.append(pltpu.async_copy(h_hbm.at[srcq[b]], rows[b],
                                               gsem[b]))
                for b in range(_NBUF):
                    j = g2 * _NBUF + b
                    gd[b].wait()
                    pltpu.async_copy(rows[b], acc.at[dst_all.at[j]], ssem[b],
                                     add=True)
                    if p == 0 and with_deg:
                        @pl.when(c == 0)
                        def _(j=j):
                            pltpu.async_copy(ones_v, dacc.at[dst_all.at[j]],
                                             dsem, add=True)
                return 0

            lax.fori_loop(0, _NOUTER, outer, 0)
            for b in range(_NBUF):
                pltpu.make_async_copy(
                    rows[b], acc.at[dst_all.at[_NCHUNKS - _NBUF + b]],
                    ssem[b]).wait()
            if p == 0 and with_deg:
                def ddrain(j, _):
                    @pl.when(c == 0)
                    def _():
                        pltpu.make_async_copy(ones_v, dacc.at[dst_all.at[j]],
                                              dsem).wait()
                    return 0

                lax.fori_loop(0, _NCHUNKS, ddrain, 0)
            plsc.subcore_barrier()

            pltpu.sync_copy(
                acc.at[pl.ds(r0, _ROWS_PER_TILE)],
                out_hbm.at[pl.ds(q * _N + r0, _ROWS_PER_TILE)])

            @pl.when(s == _NTILES - 1)
            def _():
                pltpu.sync_copy(acc.at[pl.ds(tail0, _TAIL_ROWS)],
                                out_hbm.at[pl.ds(q * _N + tail0, _TAIL_ROWS)])

            if p == 0 and with_deg:
                @pl.when(c == 0)
                def _():
                    pltpu.sync_copy(dacc.at[pl.ds(r0, _ROWS_PER_TILE)],
                                    deg_hbm.at[pl.ds(r0, _ROWS_PER_TILE)])

                @pl.when((c == 0) & (s == _NTILES - 1))
                def _():
                    pltpu.sync_copy(dacc.at[pl.ds(tail0, _TAIL_ROWS)],
                                    deg_hbm.at[pl.ds(tail0, _TAIL_ROWS)])

    return body(hq, src_t, dst_t)


def _fused_body(t_in_ref, a4_ref, sum_in_ref, sq_in_ref, g_ref, be_ref,
                w1_ref, b1_ref, w2_ref, b2_ref, ep_ref, deg_ref, batch_ref,
                t_ref, sum_ref, sq_ref, pool_ref, cnt_ref):
    i = pl.program_id(0)
    inv_n = 1.0 / _N
    mean = sum_in_ref[0:1, :] * inv_n
    var = sq_in_ref[0:1, :] * inv_n - mean * mean
    a = lax.rsqrt(var + 1e-5) * g_ref[...]
    bb = be_ref[...] - mean * a
    ep = ep_ref[0, 0]
    u = t_in_ref[...] * ep + jnp.concatenate(
        [a4_ref[k] for k in range(_NQ)], axis=1)
    w1eff = jnp.reshape(a, (_D, 1)) * w1_ref[...]
    bvec = jnp.dot(bb, w1_ref[...], preferred_element_type=jnp.float32)
    dvec = jnp.reshape(deg_ref[0, 0], (_BLK, 1)) + ep
    z = jnp.maximum(
        jnp.dot(u, w1eff, preferred_element_type=jnp.float32)
        + dvec * bvec + b1_ref[...], 0.0)
    z = jnp.maximum(
        jnp.dot(z, w2_ref[...], preferred_element_type=jnp.float32)
        + b2_ref[...], 0.0)
    t_ref[...] = z
    ps = jnp.broadcast_to(jnp.sum(z, axis=0, keepdims=True), (8, _D))
    pq = jnp.broadcast_to(jnp.sum(z * z, axis=0, keepdims=True), (8, _D))
    b = batch_ref[0, 0]
    oh = (b[:, None] == lax.broadcasted_iota(jnp.int32, (_BLK, _G), 1)
          ).astype(jnp.float32)
    pp = lax.dot_general(oh, z, (((0,), (0,)), ((), ())),
                         preferred_element_type=jnp.float32)
    pc = lax.dot_general(oh, jnp.ones((_BLK, 128), jnp.float32),
                         (((0,), (0,)), ((), ())),
                         preferred_element_type=jnp.float32)

    @pl.when(i == 0)
    def _():
        sum_ref[...] = jnp.zeros_like(sum_ref)
        sq_ref[...] = jnp.zeros_like(sq_ref)
        pool_ref[...] = jnp.zeros_like(pool_ref)
        cnt_ref[...] = jnp.zeros_like(cnt_ref)

    sum_ref[...] += ps
    sq_ref[...] += pq
    pool_ref[...] += pp
    cnt_ref[...] += pc


def _pool_affine(pt, ss, qq, gg, ee, cnt1):
    mean = ss[0:1, :] * (1.0 / _N)
    var = qq[0:1, :] * (1.0 / _N) - mean * mean
    a = lax.rsqrt(var + 1e-5) * gg
    bb = ee - mean * a
    return pt * a + cnt1 * bb


def _fused_head_body(*refs):
    (p0, p1, p2,
     s0, q0, g0, e0, s1, q1, g1, e1, s2, q2, g2, e2,
     g3, e3, hw1, hb1, hw2, hb2, hw3, hb3, hw4, hb4) = refs[13:38]
    t_ref, sum_ref, sq_ref, pool_ref, cnt_ref, out_ref = refs[38:]
    _fused_body(*(refs[:13] + (t_ref, sum_ref, sq_ref, pool_ref, cnt_ref)))

    @pl.when(pl.program_id(0) == _NBLK - 1)
    def _():
        cnt1 = cnt_ref[:, 0:1]
        phs = [_pool_affine(pt[...], ss[...], qq[...], gg[...], ee[...], cnt1)
               for pt, ss, qq, gg, ee in
               ((p0, s0, q0, g0, e0), (p1, s1, q1, g1, e1),
                (p2, s2, q2, g2, e2))]
        phs.append(_pool_affine(pool_ref[...], sum_ref[...], sq_ref[...],
                                g3[...], e3[...], cnt1))
        ge = jnp.concatenate(phs, axis=1)
        ge = ge / jnp.maximum(cnt1, 1.0)
        g = jnp.maximum(
            jnp.dot(ge, hw1[...], preferred_element_type=jnp.float32)
            + hb1[...], 0.0)
        g = jnp.maximum(
            jnp.dot(g, hw2[...], preferred_element_type=jnp.float32)
            + hb2[...], 0.0)
        g = jnp.maximum(
            jnp.dot(g, hw3[...], preferred_element_type=jnp.float32)
            + hb3[...], 0.0)
        lg = jnp.dot(g, hw4[...], preferred_element_type=jnp.float32) + hb4[...]
        m = jnp.max(lg, axis=1, keepdims=True)
        e = jnp.exp(lg - m)
        out_ref[...] = lg - m - jnp.log(jnp.sum(e, axis=1, keepdims=True))


def _fused(t_in, a4, ssum_in, sq_in, gamma, beta, w1, b1, w2, b2, onep,
           deg_r, batch_r, head=None):
    full = lambda i: (0, 0)
    in_specs = [
        pl.BlockSpec((_BLK, _D), lambda i: (i, 0)),
        pl.BlockSpec((_NQ, _BLK, _Q), lambda i: (0, i, 0)),
        pl.BlockSpec((8, _D), full),
        pl.BlockSpec((8, _D), full),
        pl.BlockSpec((1, _D), full),
        pl.BlockSpec((1, _D), full),
        pl.BlockSpec((_D, _D), full),
        pl.BlockSpec((1, _D), full),
        pl.BlockSpec((_D, _D), full),
        pl.BlockSpec((1, _D), full),
        pl.BlockSpec((1, 1), full),
        pl.BlockSpec((1, 1, _BLK), lambda i: (i, 0, 0)),
        pl.BlockSpec((1, 1, _BLK), lambda i: (i, 0, 0)),
    ]
    out_specs = [
        pl.BlockSpec((_BLK, _D), lambda i: (i, 0)),
        pl.BlockSpec((8, _D), full),
        pl.BlockSpec((8, _D), full),
        pl.BlockSpec((_G, _D), full),
        pl.BlockSpec((_G, 128), full),
    ]
    out_shape = [
        jax.ShapeDtypeStruct((_N, _D), jnp.float32),
        jax.ShapeDtypeStruct((8, _D), jnp.float32),
        jax.ShapeDtypeStruct((8, _D), jnp.float32),
        jax.ShapeDtypeStruct((_G, _D), jnp.float32),
        jax.ShapeDtypeStruct((_G, 128), jnp.float32),
    ]
    args = [t_in, a4, ssum_in, sq_in, gamma, beta, w1, b1, w2, b2, onep,
            deg_r, batch_r]
    body = _fused_body
    if head is not None:
        in_specs = in_specs + [pl.BlockSpec(h.shape, full) for h in head]
        args = args + list(head)
        out_specs = out_specs + [pl.BlockSpec((_G, 128), full)]
        out_shape = out_shape + [jax.ShapeDtypeStruct((_G, 128), jnp.float32)]
        body = _fused_head_body
    return pl.pallas_call(
        body,
        grid=(_NBLK,),
        in_specs=in_specs,
        out_specs=out_specs,
        out_shape=out_shape,
        compiler_params=pltpu.CompilerParams(
            dimension_semantics=("arbitrary",)),
    )(*args)


def kernel(x, edge_index, batch, params):
    src = edge_index[0]
    dst = edge_index[1]
    npad = _EPAD - _E
    src_t = jnp.pad(src * _NQ, (0, npad)).reshape(_NTILES, _NCHUNKS, _CHUNK)
    dst_t = jnp.pad(dst, (0, npad), constant_values=_N).reshape(
        _NTILES, _NCHUNKS, _CHUNK)
    batch_r = batch.reshape(_NBLK, 1, _BLK)

    ssum = jnp.zeros((8, _D), jnp.float32)
    sq = jnp.full((8, _D), _N * (1.0 - 1e-5), jnp.float32)
    gamma = jnp.ones((1, _D), jnp.float32)
    beta = jnp.zeros((1, _D), jnp.float32)

    t = x
    pools = []
    stats = []
    deg_r = None
    for li in range(_L):
        p = params['gin%d' % li]
        if li == 0:
            aggr, deg = _sc_aggregate(t.reshape(_NQ * _N, _Q), src_t, dst_t,
                                      with_deg=True)
            deg_r = deg[:, 0].reshape(_NBLK, 1, _BLK)
        else:
            aggr = _sc_aggregate(t.reshape(_NQ * _N, _Q), src_t, dst_t)
        a4 = aggr.reshape(_NQ, _N, _Q)
        onep = (1.0 + p['eps']).reshape(1, 1)
        head = None
        if li == _L - 1:
            c = params['cls']
            nc = c['W4'].shape[1]
            w4p = jnp.pad(c['W4'], ((0, 0), (0, 128 - nc)))
            b4p = jnp.pad(c['b4'].reshape(1, -1), ((0, 0), (0, 128 - nc)),
                          constant_values=-1e30)
            flat_stats = []
            for st in stats:
                flat_stats.extend(st)
            head = (pools + flat_stats
                    + [p['gamma'].reshape(1, -1), p['beta'].reshape(1, -1),
                       c['W1'], c['b1'].reshape(1, -1),
                       c['W2'], c['b2'].reshape(1, -1),
                       c['W3'], c['b3'].reshape(1, -1), w4p, b4p])
        outs = _fused(
            t, a4, ssum, sq, gamma, beta,
            p['W1'], p['b1'].reshape(1, -1),
            p['W2'], p['b2'].reshape(1, -1), onep, deg_r, batch_r, head=head)
        t, ssum, sq, pool_i = outs[0], outs[1], outs[2], outs[3]
        gamma = p['gamma'].reshape(1, -1)
        beta = p['beta'].reshape(1, -1)
        pools.append(pool_i)
        stats.append((ssum, sq, gamma, beta))

    return outs[5][:, :params['cls']['W4'].shape[1]]
